# baseline (speedup 1.0000x reference)
import jax
import jax.numpy as jnp
from jax import lax
from jax.experimental import pallas as pl
from jax.experimental.pallas import tpu as pltpu
from jax.experimental.pallas import tpu_sc as plsc

B, L, D, T = 16, 4096, 512, 32768
C = 64
NCH = L // C
KPW = NCH // 2


def _sc_body(flat, cu_pad, out, cu_v, buf0, buf1, rd0, rd1, wr0, wr1, sem_r):
    wid = lax.axis_index("c") * 16 + lax.axis_index("s")
    b = wid // 2
    h = wid % 2
    rowbase = b * L

    pltpu.sync_copy(cu_pad, cu_v)

    v = cu_v[pl.ds(b, 16)]
    cu_b = v[0]
    seg_len = jnp.clip(v[1] - cu_b, 0, L)
    nfb = seg_len // C
    p = seg_len - nfb * C

    bufs = (buf0, buf1)
    rds = (rd0, rd1)
    wrs = (wr0, wr1)

    def src(k):
        return flat.at[pl.ds((cu_b + (2 * k + h) * C) * D, C * D)]

    def dst(k):
        return out.at[pl.ds((rowbase + (2 * k + h) * C) * D, C * D)]

    nc = jnp.clip((nfb - h + 1) // 2, 0, KPW)
    has_bnd = jnp.logical_and(p > 0, nfb % 2 == h)

    @pl.when(has_bnd)
    def _():
        def row_body(j, carry):
            pltpu.async_copy(
                flat.at[pl.ds((cu_b + nfb * C + j) * D, D)],
                out.at[pl.ds((rowbase + nfb * C + j) * D, D)], sem_r)
            return carry

        lax.fori_loop(0, p, row_body, 0)

    for j in range(2):
        @pl.when(nc > j)
        def _():
            pltpu.async_copy(src(j), bufs[j], rds[j])

    def pipe_body(k2, carry):
        for j in range(2):
            k = 2 * k2 + j

            @pl.when(k < nc)
            def _():
                pltpu.make_async_copy(flat.at[pl.ds(0, C * D)],
                                      bufs[j], rds[j]).wait()
                pltpu.async_copy(bufs[j], dst(k), wrs[j])

                @pl.when(k + 2 < nc)
                def _():
                    pltpu.make_async_copy(bufs[j], out.at[pl.ds(0, C * D)],
                                          wrs[j]).wait()
                    pltpu.async_copy(src(k + 2), bufs[j], rds[j])

        return carry

    lax.fori_loop(0, (nc + 1) // 2, pipe_body, 0)

    for j in range(2):
        @pl.when(nc > j)
        def _():
            pltpu.make_async_copy(bufs[j], out.at[pl.ds(0, C * D)],
                                  wrs[j]).wait()

    @pl.when(has_bnd)
    def _():
        def drain_r(_, carry):
            pltpu.make_async_copy(flat.at[pl.ds(0, D)],
                                  out.at[pl.ds(0, D)], sem_r).wait()
            return carry

        lax.fori_loop(0, p, drain_r, 0)


def _tc_body(cu_ref, in_ref, out_ref, zbuf, sem_r, sem_z):
    del in_ref
    zbuf[...] = jnp.zeros((C * D,), jnp.float32)

    nrow = jnp.int32(0)
    nchk = jnp.int32(0)
    for b in range(B):
        seg_len = jnp.clip(cu_ref[b + 1] - cu_ref[b], 0, L)
        nfb = seg_len // C
        p = seg_len - nfb * C
        rowbase = b * L

        def row_body(j, carry):
            pltpu.async_copy(zbuf.at[pl.ds(0, D)],
                             out_ref.at[pl.ds((rowbase + nfb * C + j) * D, D)],
                             sem_r)
            return carry

        lax.fori_loop(p, jnp.where(p > 0, C, 0), row_body, 0)

        z0 = nfb + (p > 0).astype(jnp.int32)

        def chunk_body(i, carry):
            pltpu.async_copy(zbuf,
                             out_ref.at[pl.ds((rowbase + i * C) * D, C * D)],
                             sem_z)
            return carry

        lax.fori_loop(z0, NCH, chunk_body, 0)

        nrow = nrow + jnp.where(p > 0, C - p, 0)
        nchk = nchk + (NCH - z0)

    def drain_r(_, carry):
        pltpu.make_async_copy(zbuf.at[pl.ds(0, D)],
                              out_ref.at[pl.ds(0, D)], sem_r).wait()
        return carry

    def drain_z(_, carry):
        pltpu.make_async_copy(zbuf, out_ref.at[pl.ds(0, C * D)],
                              sem_z).wait()
        return carry

    lax.fori_loop(0, nrow, drain_r, 0)
    lax.fori_loop(0, nchk, drain_z, 0)


def kernel(flat, cu_seqlens):
    cu = cu_seqlens.astype(jnp.int32)
    cu_pad = jnp.zeros((2 * B,), jnp.int32).at[:B + 1].set(cu)
    mesh = plsc.VectorSubcoreMesh(core_axis_name="c", subcore_axis_name="s")
    sc_run = pl.kernel(
        _sc_body,
        mesh=mesh,
        out_type=jax.ShapeDtypeStruct((B * L * D,), jnp.float32),
        scratch_types=[
            pltpu.VMEM((2 * B,), jnp.int32),
            pltpu.VMEM((C * D,), jnp.float32),
            pltpu.VMEM((C * D,), jnp.float32),
            pltpu.SemaphoreType.DMA,
            pltpu.SemaphoreType.DMA,
            pltpu.SemaphoreType.DMA,
            pltpu.SemaphoreType.DMA,
            pltpu.SemaphoreType.DMA,
        ],
    )
    draft = sc_run(flat.reshape(T * D), cu_pad)

    dense = pl.pallas_call(
        _tc_body,
        out_shape=jax.ShapeDtypeStruct((B * L * D,), jnp.float32),
        in_specs=[
            pl.BlockSpec(memory_space=pltpu.SMEM),
            pl.BlockSpec(memory_space=pl.ANY),
        ],
        out_specs=pl.BlockSpec(memory_space=pl.ANY),
        scratch_shapes=[
            pltpu.VMEM((C * D,), jnp.float32),
            pltpu.SemaphoreType.DMA,
            pltpu.SemaphoreType.DMA,
        ],
        input_output_aliases={1: 0},
    )(cu, draft)
    return dense.reshape(B, L, D)


# speedup vs baseline: 1.4450x; 1.4450x over previous
import jax
import jax.numpy as jnp
from jax import lax
from jax.experimental import pallas as pl
from jax.experimental.pallas import tpu as pltpu
from jax.experimental.pallas import tpu_sc as plsc

B, L, D, T = 16, 4096, 512, 32768
C = 64
LP = L + C
G = 512
WG = G + 8
NG = L // G
NT = B * NG


def _sc_body(flat, cu_pad, draft, cu_v, buf0, buf1, rd0, rd1, wr0, wr1):
    wid = lax.axis_index("c") * 16 + lax.axis_index("s")
    b = wid // 2
    h = wid % 2
    rowbase = b * LP

    pltpu.sync_copy(cu_pad, cu_v)

    v = cu_v[pl.ds(b, 16)]
    cu_b = v[0]
    seg_len = jnp.clip(v[1] - cu_b, 0, L)
    s = lax.rem(cu_b, 8)
    a0 = cu_b - s
    nsc = (seg_len + s + C - 1) // C

    bufs = (buf0, buf1)
    rds = (rd0, rd1)
    wrs = (wr0, wr1)

    def st_of(k):
        return jnp.minimum(a0 + (2 * k + h) * C, T - C)

    def src(k):
        return flat.at[pl.ds(pl.multiple_of(st_of(k), 8), C)]

    def dst(k):
        off = rowbase + (st_of(k) - a0)
        return draft.at[pl.ds(pl.multiple_of(off, 8), C)]

    nc = jnp.clip((nsc - h + 1) // 2, 0, LP // (2 * C) + 1)

    for j in range(2):
        @pl.when(nc > j)
        def _():
            pltpu.async_copy(src(j), bufs[j], rds[j])

    def pipe_body(k2, carry):
        for j in range(2):
            k = 2 * k2 + j

            @pl.when(k < nc)
            def _():
                pltpu.make_async_copy(flat.at[pl.ds(0, C)],
                                      bufs[j], rds[j]).wait()
                pltpu.async_copy(bufs[j], dst(k), wrs[j])

                @pl.when(k + 2 < nc)
                def _():
                    pltpu.make_async_copy(bufs[j], draft.at[pl.ds(0, C)],
                                          wrs[j]).wait()
                    pltpu.async_copy(src(k + 2), bufs[j], rds[j])

        return carry

    lax.fori_loop(0, (nc + 1) // 2, pipe_body, 0)

    for j in range(2):
        @pl.when(nc > j)
        def _():
            pltpu.make_async_copy(bufs[j], draft.at[pl.ds(0, C)],
                                  wrs[j]).wait()


def _tc_body(cu_ref, draft, out_ref, w0, w1, ob0, ob1, zb,
             rd0, rd1, wr0, wr1, sem_z):
    ws = (w0, w1)
    obs = (ob0, ob1)
    rds = (rd0, rd1)
    wrs = (wr0, wr1)

    zb[...] = jnp.zeros((G, D), jnp.float32)

    def row_len(bi):
        return jnp.clip(cu_ref[bi + 1] - cu_ref[bi], 0, L)

    def cond(t):
        return lax.rem(t, NG) < (row_len(t // NG) + G - 1) // G

    def win_ref(t):
        off = (t // NG) * LP + lax.rem(t, NG) * G
        return draft.at[pl.ds(pl.multiple_of(off, 8), WG)]

    def out_at(t):
        return out_ref.at[pl.ds(pl.multiple_of(t * G, 8), G)]

    tz = jnp.int32(NT)
    for bi in range(B):
        tz = tz - (row_len(bi) + G - 1) // G

    for j in range(2):
        @pl.when(cond(j))
        def _():
            pltpu.async_copy(win_ref(j), ws[j], rds[j])

    def body(t2, carry):
        for j in range(2):
            t = 2 * t2 + j

            @pl.when(jnp.logical_and(t >= 2, cond(jnp.maximum(t - 2, 0))))
            def _():
                pltpu.make_async_copy(obs[j], out_ref.at[pl.ds(0, G)],
                                      wrs[j]).wait()

            @pl.when(cond(t))
            def _():
                pltpu.make_async_copy(draft.at[pl.ds(0, WG)],
                                      ws[j], rds[j]).wait()
                bi = t // NG
                g = lax.rem(t, NG)
                ln = row_len(bi)
                s = lax.rem(cu_ref[bi], 8)
                rolled = pltpu.roll(ws[j][...], WG - s, 0)[:G]
                partial = g * G + G > ln

                @pl.when(partial)
                def _():
                    rows = lax.broadcasted_iota(jnp.int32, (G, D), 0) + g * G
                    obs[j][...] = jnp.where(rows < ln, rolled, 0.0)

                @pl.when(jnp.logical_not(partial))
                def _():
                    obs[j][...] = rolled

                pltpu.async_copy(obs[j], out_at(t), wrs[j])

            @pl.when(jnp.logical_not(cond(t)))
            def _():
                pltpu.async_copy(zb, out_at(t), sem_z)

            tn = jnp.minimum(t + 2, NT - 1)

            @pl.when(jnp.logical_and(t + 2 < NT, cond(tn)))
            def _():
                pltpu.async_copy(win_ref(tn), ws[j], rds[j])

        return carry

    lax.fori_loop(0, NT // 2, body, 0)

    for t in (NT - 2, NT - 1):
        @pl.when(cond(jnp.int32(t)))
        def _():
            pltpu.make_async_copy(obs[t % 2], out_ref.at[pl.ds(0, G)],
                                  wrs[t % 2]).wait()

    def drain_z(_, carry):
        pltpu.make_async_copy(zb, out_ref.at[pl.ds(0, G)], sem_z).wait()
        return carry

    lax.fori_loop(0, tz, drain_z, 0)


def kernel(flat, cu_seqlens):
    cu = cu_seqlens.astype(jnp.int32)
    cu_pad = jnp.zeros((2 * B,), jnp.int32).at[:B + 1].set(cu)
    mesh = plsc.VectorSubcoreMesh(core_axis_name="c", subcore_axis_name="s")
    sc_run = pl.kernel(
        _sc_body,
        mesh=mesh,
        out_type=jax.ShapeDtypeStruct((B * LP, D), jnp.float32),
        scratch_types=[
            pltpu.VMEM((2 * B,), jnp.int32),
            pltpu.VMEM((C, D), jnp.float32),
            pltpu.VMEM((C, D), jnp.float32),
            pltpu.SemaphoreType.DMA,
            pltpu.SemaphoreType.DMA,
            pltpu.SemaphoreType.DMA,
            pltpu.SemaphoreType.DMA,
        ],
    )
    draft = sc_run(flat, cu_pad)

    dense = pl.pallas_call(
        _tc_body,
        out_shape=jax.ShapeDtypeStruct((B * L, D), jnp.float32),
        in_specs=[
            pl.BlockSpec(memory_space=pltpu.SMEM),
            pl.BlockSpec(memory_space=pl.ANY),
        ],
        out_specs=pl.BlockSpec(memory_space=pl.ANY),
        scratch_shapes=[
            pltpu.VMEM((WG, D), jnp.float32),
            pltpu.VMEM((WG, D), jnp.float32),
            pltpu.VMEM((G, D), jnp.float32),
            pltpu.VMEM((G, D), jnp.float32),
            pltpu.VMEM((G, D), jnp.float32),
            pltpu.SemaphoreType.DMA,
            pltpu.SemaphoreType.DMA,
            pltpu.SemaphoreType.DMA,
            pltpu.SemaphoreType.DMA,
            pltpu.SemaphoreType.DMA,
        ],
    )(cu, draft)
    return dense.reshape(B, L, D)


# speedup vs baseline: 1.6194x; 1.1207x over previous
import jax
import jax.numpy as jnp
from jax import lax
from jax.experimental import pallas as pl
from jax.experimental.pallas import tpu as pltpu
from jax.experimental.pallas import tpu_sc as plsc

B, L, D, T = 16, 4096, 512, 32768
C = 64
LP = L + C
G = 512
WG = G + 8
NG = L // G
NT = B * NG


def _sc_body(flat, cu_pad, draft, cu_v, buf0, buf1, rd0, rd1, wr0, wr1):
    wid = lax.axis_index("c") * 16 + lax.axis_index("s")
    b = wid // 2
    h = wid % 2
    rowbase = b * LP

    pltpu.sync_copy(cu_pad, cu_v)

    v = cu_v[pl.ds(b, 16)]
    cu_b = v[0]
    seg_len = jnp.clip(v[1] - cu_b, 0, L)
    s = lax.rem(cu_b, 8)
    a0 = cu_b - s
    nsc = (seg_len + s + C - 1) // C

    bufs = (buf0, buf1)
    rds = (rd0, rd1)
    wrs = (wr0, wr1)

    def st_of(k):
        return jnp.minimum(a0 + (2 * k + h) * C, T - C)

    def src(k):
        return flat.at[pl.ds(pl.multiple_of(st_of(k), 8), C)]

    def dst(k):
        off = rowbase + (st_of(k) - a0)
        return draft.at[pl.ds(pl.multiple_of(off, 8), C)]

    nc = jnp.clip((nsc - h + 1) // 2, 0, LP // (2 * C) + 1)

    for j in range(2):
        @pl.when(nc > j)
        def _():
            pltpu.async_copy(src(j), bufs[j], rds[j])

    def pipe_body(k2, carry):
        for j in range(2):
            k = 2 * k2 + j

            @pl.when(k < nc)
            def _():
                pltpu.make_async_copy(flat.at[pl.ds(0, C)],
                                      bufs[j], rds[j]).wait()
                pltpu.async_copy(bufs[j], dst(k), wrs[j])

                @pl.when(k + 2 < nc)
                def _():
                    pltpu.make_async_copy(bufs[j], draft.at[pl.ds(0, C)],
                                          wrs[j]).wait()
                    pltpu.async_copy(src(k + 2), bufs[j], rds[j])

        return carry

    lax.fori_loop(0, (nc + 1) // 2, pipe_body, 0)

    for j in range(2):
        @pl.when(nc > j)
        def _():
            pltpu.make_async_copy(bufs[j], draft.at[pl.ds(0, C)],
                                  wrs[j]).wait()


def _tc_body(cu_ref, draft, out_ref, w0, w1, ob0, ob1, zb,
             rd0, rd1, wr0, wr1, sem_z):
    ws = (w0, w1)
    obs = (ob0, ob1)
    rds = (rd0, rd1)
    wrs = (wr0, wr1)

    zb[...] = jnp.zeros((G, D), jnp.float32)

    def row_len(bi):
        return jnp.clip(cu_ref[bi + 1] - cu_ref[bi], 0, L)

    def cond(t):
        return lax.rem(t, NG) < (row_len(t // NG) + G - 1) // G

    def win_ref(t):
        off = (t // NG) * LP + lax.rem(t, NG) * G
        return draft.at[pl.ds(pl.multiple_of(off, 8), WG)]

    def out_at(t):
        return out_ref.at[pl.ds(pl.multiple_of(t * G, 8), G)]

    tz = jnp.int32(NT)
    for bi in range(B):
        tz = tz - (row_len(bi) + G - 1) // G

    for j in range(2):
        @pl.when(cond(j))
        def _():
            pltpu.async_copy(win_ref(j), ws[j], rds[j])

    def body(t2, carry):
        for j in range(2):
            t = 2 * t2 + j

            @pl.when(jnp.logical_and(t >= 2, cond(jnp.maximum(t - 2, 0))))
            def _():
                pltpu.make_async_copy(obs[j].at[pl.ds(0, G)],
                                      out_ref.at[pl.ds(0, G)], wrs[j]).wait()

            @pl.when(cond(t))
            def _():
                pltpu.make_async_copy(draft.at[pl.ds(0, WG)],
                                      ws[j], rds[j]).wait()
                bi = t // NG
                g = lax.rem(t, NG)
                ln = row_len(bi)
                s = lax.rem(cu_ref[bi], 8)
                for sv in range(8):
                    @pl.when(s == sv)
                    def _(sv=sv):
                        if sv == 0:
                            obs[j][...] = ws[j][...]
                        else:
                            obs[j][...] = pltpu.roll(ws[j][...], WG - sv, 0)

                partial = g * G + G > ln

                @pl.when(partial)
                def _():
                    rows = lax.broadcasted_iota(jnp.int32, (WG, D), 0) + g * G
                    obs[j][...] = jnp.where(rows < ln, obs[j][...], 0.0)

                pltpu.async_copy(obs[j].at[pl.ds(0, G)], out_at(t), wrs[j])

            @pl.when(jnp.logical_not(cond(t)))
            def _():
                pltpu.async_copy(zb, out_at(t), sem_z)

            tn = jnp.minimum(t + 2, NT - 1)

            @pl.when(jnp.logical_and(t + 2 < NT, cond(tn)))
            def _():
                pltpu.async_copy(win_ref(tn), ws[j], rds[j])

        return carry

    lax.fori_loop(0, NT // 2, body, 0)

    for t in (NT - 2, NT - 1):
        @pl.when(cond(jnp.int32(t)))
        def _():
            pltpu.make_async_copy(obs[t % 2].at[pl.ds(0, G)],
                                  out_ref.at[pl.ds(0, G)], wrs[t % 2]).wait()

    def drain_z(_, carry):
        pltpu.make_async_copy(zb, out_ref.at[pl.ds(0, G)], sem_z).wait()
        return carry

    lax.fori_loop(0, tz, drain_z, 0)


def kernel(flat, cu_seqlens):
    cu = cu_seqlens.astype(jnp.int32)
    cu_pad = jnp.zeros((2 * B,), jnp.int32).at[:B + 1].set(cu)
    mesh = plsc.VectorSubcoreMesh(core_axis_name="c", subcore_axis_name="s")
    sc_run = pl.kernel(
        _sc_body,
        mesh=mesh,
        out_type=jax.ShapeDtypeStruct((B * LP, D), jnp.float32),
        scratch_types=[
            pltpu.VMEM((2 * B,), jnp.int32),
            pltpu.VMEM((C, D), jnp.float32),
            pltpu.VMEM((C, D), jnp.float32),
            pltpu.SemaphoreType.DMA,
            pltpu.SemaphoreType.DMA,
            pltpu.SemaphoreType.DMA,
            pltpu.SemaphoreType.DMA,
        ],
    )
    draft = sc_run(flat, cu_pad)

    dense = pl.pallas_call(
        _tc_body,
        out_shape=jax.ShapeDtypeStruct((B * L, D), jnp.float32),
        in_specs=[
            pl.BlockSpec(memory_space=pltpu.SMEM),
            pl.BlockSpec(memory_space=pl.ANY),
        ],
        out_specs=pl.BlockSpec(memory_space=pl.ANY),
        scratch_shapes=[
            pltpu.VMEM((WG, D), jnp.float32),
            pltpu.VMEM((WG, D), jnp.float32),
            pltpu.VMEM((WG, D), jnp.float32),
            pltpu.VMEM((WG, D), jnp.float32),
            pltpu.VMEM((G, D), jnp.float32),
            pltpu.SemaphoreType.DMA,
            pltpu.SemaphoreType.DMA,
            pltpu.SemaphoreType.DMA,
            pltpu.SemaphoreType.DMA,
            pltpu.SemaphoreType.DMA,
        ],
    )(cu, draft)
    return dense.reshape(B, L, D)


# speedup vs baseline: 1.7155x; 1.0593x over previous
import jax
import jax.numpy as jnp
from jax import lax
from jax.experimental import pallas as pl
from jax.experimental.pallas import tpu as pltpu
from jax.experimental.pallas import tpu_sc as plsc

B, L, D, T = 16, 4096, 512, 32768
C = 64
LP = L + C
G = 1024
WG = G + 8
NG = L // G
NT = B * NG


def _sc_body(flat, cu_pad, draft, cu_v, buf0, buf1, rd0, rd1, wr0, wr1):
    wid = lax.axis_index("c") * 16 + lax.axis_index("s")
    b = wid // 2
    h = wid % 2
    rowbase = b * LP

    pltpu.sync_copy(cu_pad, cu_v)

    v = cu_v[pl.ds(b, 16)]
    cu_b = v[0]
    seg_len = jnp.clip(v[1] - cu_b, 0, L)
    s = lax.rem(cu_b, 8)
    a0 = cu_b - s
    nsc = (seg_len + s + C - 1) // C

    bufs = (buf0, buf1)
    rds = (rd0, rd1)
    wrs = (wr0, wr1)

    def st_of(k):
        return jnp.minimum(a0 + (2 * k + h) * C, T - C)

    def src(k):
        return flat.at[pl.ds(pl.multiple_of(st_of(k), 8), C)]

    def dst(k):
        off = rowbase + (st_of(k) - a0)
        return draft.at[pl.ds(pl.multiple_of(off, 8), C)]

    nc = jnp.clip((nsc - h + 1) // 2, 0, LP // (2 * C) + 1)

    for j in range(2):
        @pl.when(nc > j)
        def _():
            pltpu.async_copy(src(j), bufs[j], rds[j])

    def pipe_body(k2, carry):
        for j in range(2):
            k = 2 * k2 + j

            @pl.when(k < nc)
            def _():
                pltpu.make_async_copy(flat.at[pl.ds(0, C)],
                                      bufs[j], rds[j]).wait()
                pltpu.async_copy(bufs[j], dst(k), wrs[j])

                @pl.when(k + 2 < nc)
                def _():
                    pltpu.make_async_copy(bufs[j], draft.at[pl.ds(0, C)],
                                          wrs[j]).wait()
                    pltpu.async_copy(src(k + 2), bufs[j], rds[j])

        return carry

    lax.fori_loop(0, (nc + 1) // 2, pipe_body, 0)

    for j in range(2):
        @pl.when(nc > j)
        def _():
            pltpu.make_async_copy(bufs[j], draft.at[pl.ds(0, C)],
                                  wrs[j]).wait()


def _tc_body(cu_ref, draft, out_ref, w0, w1, ob0, ob1, zb,
             rd0, rd1, wr0, wr1, sem_z):
    ws = (w0, w1)
    obs = (ob0, ob1)
    rds = (rd0, rd1)
    wrs = (wr0, wr1)

    zb[...] = jnp.zeros((G, D), jnp.float32)

    def row_len(bi):
        return jnp.clip(cu_ref[bi + 1] - cu_ref[bi], 0, L)

    def cond(t):
        return lax.rem(t, NG) < (row_len(t // NG) + G - 1) // G

    def win_ref(t):
        off = (t // NG) * LP + lax.rem(t, NG) * G
        return draft.at[pl.ds(pl.multiple_of(off, 8), WG)]

    def out_at(t):
        return out_ref.at[pl.ds(pl.multiple_of(t * G, 8), G)]

    tz = jnp.int32(NT)
    for bi in range(B):
        tz = tz - (row_len(bi) + G - 1) // G

    for j in range(2):
        @pl.when(cond(j))
        def _():
            pltpu.async_copy(win_ref(j), ws[j], rds[j])

    def body(t2, carry):
        for j in range(2):
            t = 2 * t2 + j

            @pl.when(jnp.logical_and(t >= 2, cond(jnp.maximum(t - 2, 0))))
            def _():
                pltpu.make_async_copy(obs[j].at[pl.ds(0, G)],
                                      out_ref.at[pl.ds(0, G)], wrs[j]).wait()

            @pl.when(cond(t))
            def _():
                pltpu.make_async_copy(draft.at[pl.ds(0, WG)],
                                      ws[j], rds[j]).wait()
                bi = t // NG
                g = lax.rem(t, NG)
                ln = row_len(bi)
                s = lax.rem(cu_ref[bi], 8)
                for sv in range(8):
                    @pl.when(s == sv)
                    def _(sv=sv):
                        if sv == 0:
                            obs[j][...] = ws[j][...]
                        else:
                            obs[j][...] = pltpu.roll(ws[j][...], WG - sv, 0)

                partial = g * G + G > ln

                @pl.when(partial)
                def _():
                    rows = lax.broadcasted_iota(jnp.int32, (WG, D), 0) + g * G
                    obs[j][...] = jnp.where(rows < ln, obs[j][...], 0.0)

                pltpu.async_copy(obs[j].at[pl.ds(0, G)], out_at(t), wrs[j])

            @pl.when(jnp.logical_not(cond(t)))
            def _():
                pltpu.async_copy(zb, out_at(t), sem_z)

            tn = jnp.minimum(t + 2, NT - 1)

            @pl.when(jnp.logical_and(t + 2 < NT, cond(tn)))
            def _():
                pltpu.async_copy(win_ref(tn), ws[j], rds[j])

        return carry

    lax.fori_loop(0, NT // 2, body, 0)

    for t in (NT - 2, NT - 1):
        @pl.when(cond(jnp.int32(t)))
        def _():
            pltpu.make_async_copy(obs[t % 2].at[pl.ds(0, G)],
                                  out_ref.at[pl.ds(0, G)], wrs[t % 2]).wait()

    def drain_z(_, carry):
        pltpu.make_async_copy(zb, out_ref.at[pl.ds(0, G)], sem_z).wait()
        return carry

    lax.fori_loop(0, tz, drain_z, 0)


def kernel(flat, cu_seqlens):
    cu = cu_seqlens.astype(jnp.int32)
    cu_pad = jnp.zeros((2 * B,), jnp.int32).at[:B + 1].set(cu)
    mesh = plsc.VectorSubcoreMesh(core_axis_name="c", subcore_axis_name="s")
    sc_run = pl.kernel(
        _sc_body,
        mesh=mesh,
        out_type=jax.ShapeDtypeStruct((B * LP, D), jnp.float32),
        scratch_types=[
            pltpu.VMEM((2 * B,), jnp.int32),
            pltpu.VMEM((C, D), jnp.float32),
            pltpu.VMEM((C, D), jnp.float32),
            pltpu.SemaphoreType.DMA,
            pltpu.SemaphoreType.DMA,
            pltpu.SemaphoreType.DMA,
            pltpu.SemaphoreType.DMA,
        ],
    )
    draft = sc_run(flat, cu_pad)

    dense = pl.pallas_call(
        _tc_body,
        out_shape=jax.ShapeDtypeStruct((B * L, D), jnp.float32),
        in_specs=[
            pl.BlockSpec(memory_space=pltpu.SMEM),
            pl.BlockSpec(memory_space=pl.ANY),
        ],
        out_specs=pl.BlockSpec(memory_space=pl.ANY),
        scratch_shapes=[
            pltpu.VMEM((WG, D), jnp.float32),
            pltpu.VMEM((WG, D), jnp.float32),
            pltpu.VMEM((WG, D), jnp.float32),
            pltpu.VMEM((WG, D), jnp.float32),
            pltpu.VMEM((G, D), jnp.float32),
            pltpu.SemaphoreType.DMA,
            pltpu.SemaphoreType.DMA,
            pltpu.SemaphoreType.DMA,
            pltpu.SemaphoreType.DMA,
            pltpu.SemaphoreType.DMA,
        ],
    )(cu, draft)
    return dense.reshape(B, L, D)


# speedup vs baseline: 1.7263x; 1.0063x over previous
import jax
import jax.numpy as jnp
from jax import lax
from jax.experimental import pallas as pl
from jax.experimental.pallas import tpu as pltpu
from jax.experimental.pallas import tpu_sc as plsc

B, L, D, T = 16, 4096, 512, 32768
C = 64
LP = L + C
G = 1024
WG = G + 8
NG = L // G
NT = B * NG


def _sc_body(flat, cu_pad, draft, cu_v, buf0, buf1, rd0, rd1, wr0, wr1):
    wid = lax.axis_index("c") * 16 + lax.axis_index("s")
    b = wid // 2
    h = wid % 2
    rowbase = b * LP

    pltpu.sync_copy(cu_pad, cu_v)

    v = cu_v[pl.ds(b, 16)]
    cu_b = v[0]
    seg_len = jnp.clip(v[1] - cu_b, 0, L)
    s = lax.rem(cu_b, 8)
    a0 = cu_b - s
    nsc = (seg_len + s + C - 1) // C

    bufs = (buf0, buf1)
    rds = (rd0, rd1)
    wrs = (wr0, wr1)

    def st_of(k):
        return jnp.minimum(a0 + (2 * k + h) * C, T - C)

    def src(k):
        return flat.at[pl.ds(pl.multiple_of(st_of(k), 8), C)]

    def dst(k):
        off = rowbase + (st_of(k) - a0)
        return draft.at[pl.ds(pl.multiple_of(off, 8), C)]

    nc = jnp.clip((nsc - h + 1) // 2, 0, LP // (2 * C) + 1)

    for j in range(2):
        @pl.when(nc > j)
        def _():
            pltpu.async_copy(src(j), bufs[j], rds[j])

    def pipe_body(k2, carry):
        for j in range(2):
            k = 2 * k2 + j

            @pl.when(k < nc)
            def _():
                pltpu.make_async_copy(flat.at[pl.ds(0, C)],
                                      bufs[j], rds[j]).wait()
                pltpu.async_copy(bufs[j], dst(k), wrs[j])

                @pl.when(k + 2 < nc)
                def _():
                    pltpu.make_async_copy(bufs[j], draft.at[pl.ds(0, C)],
                                          wrs[j]).wait()
                    pltpu.async_copy(src(k + 2), bufs[j], rds[j])

        return carry

    lax.fori_loop(0, (nc + 1) // 2, pipe_body, 0)

    for j in range(2):
        @pl.when(nc > j)
        def _():
            pltpu.make_async_copy(bufs[j], draft.at[pl.ds(0, C)],
                                  wrs[j]).wait()


def _tc_zero(cu_ref, out_ref, zb, sem_z):
    zb[...] = jnp.zeros((G, D), jnp.float32)

    def row_len(bi):
        return jnp.clip(cu_ref[bi + 1] - cu_ref[bi], 0, L)

    tz = jnp.int32(0)
    for bi in range(B):
        g0 = (row_len(bi) + G - 1) // G

        def zero_body(g, carry):
            pltpu.async_copy(
                zb, out_ref.at[pl.ds(pl.multiple_of(bi * L + g * G, 8), G)],
                sem_z)
            return carry

        lax.fori_loop(g0, NG, zero_body, 0)
        tz = tz + (NG - g0)

    def drain_z(_, carry):
        pltpu.make_async_copy(zb, out_ref.at[pl.ds(0, G)], sem_z).wait()
        return carry

    lax.fori_loop(0, tz, drain_z, 0)


def _tc_data(cu_ref, draft, zeroed, out_ref, w0, w1, ob0, ob1,
             rd0, rd1, wr0, wr1):
    del zeroed
    ws = (w0, w1)
    obs = (ob0, ob1)
    rds = (rd0, rd1)
    wrs = (wr0, wr1)

    def row_len(bi):
        return jnp.clip(cu_ref[bi + 1] - cu_ref[bi], 0, L)

    def cond(t):
        return lax.rem(t, NG) < (row_len(t // NG) + G - 1) // G

    def win_ref(t):
        off = (t // NG) * LP + lax.rem(t, NG) * G
        return draft.at[pl.ds(pl.multiple_of(off, 8), WG)]

    def out_at(t):
        return out_ref.at[pl.ds(pl.multiple_of(t * G, 8), G)]

    for j in range(2):
        @pl.when(cond(j))
        def _():
            pltpu.async_copy(win_ref(j), ws[j], rds[j])

    def body(t2, carry):
        for j in range(2):
            t = 2 * t2 + j

            @pl.when(jnp.logical_and(t >= 2, cond(jnp.maximum(t - 2, 0))))
            def _():
                pltpu.make_async_copy(obs[j].at[pl.ds(0, G)],
                                      out_ref.at[pl.ds(0, G)], wrs[j]).wait()

            @pl.when(cond(t))
            def _():
                pltpu.make_async_copy(draft.at[pl.ds(0, WG)],
                                      ws[j], rds[j]).wait()
                bi = t // NG
                g = lax.rem(t, NG)
                ln = row_len(bi)
                s = lax.rem(cu_ref[bi], 8)
                for sv in range(8):
                    @pl.when(s == sv)
                    def _(sv=sv):
                        if sv == 0:
                            obs[j][...] = ws[j][...]
                        else:
                            obs[j][...] = pltpu.roll(ws[j][...], WG - sv, 0)

                partial = g * G + G > ln

                @pl.when(partial)
                def _():
                    rows = lax.broadcasted_iota(jnp.int32, (WG, D), 0) + g * G
                    obs[j][...] = jnp.where(rows < ln, obs[j][...], 0.0)

                pltpu.async_copy(obs[j].at[pl.ds(0, G)], out_at(t), wrs[j])

            tn = jnp.minimum(t + 2, NT - 1)

            @pl.when(jnp.logical_and(t + 2 < NT, cond(tn)))
            def _():
                pltpu.async_copy(win_ref(tn), ws[j], rds[j])

        return carry

    lax.fori_loop(0, NT // 2, body, 0)

    for t in (NT - 2, NT - 1):
        @pl.when(cond(jnp.int32(t)))
        def _():
            pltpu.make_async_copy(obs[t % 2].at[pl.ds(0, G)],
                                  out_ref.at[pl.ds(0, G)], wrs[t % 2]).wait()


def kernel(flat, cu_seqlens):
    cu = cu_seqlens.astype(jnp.int32)
    cu_pad = jnp.zeros((2 * B,), jnp.int32).at[:B + 1].set(cu)
    mesh = plsc.VectorSubcoreMesh(core_axis_name="c", subcore_axis_name="s")
    sc_run = pl.kernel(
        _sc_body,
        mesh=mesh,
        out_type=jax.ShapeDtypeStruct((B * LP, D), jnp.float32),
        scratch_types=[
            pltpu.VMEM((2 * B,), jnp.int32),
            pltpu.VMEM((C, D), jnp.float32),
            pltpu.VMEM((C, D), jnp.float32),
            pltpu.SemaphoreType.DMA,
            pltpu.SemaphoreType.DMA,
            pltpu.SemaphoreType.DMA,
            pltpu.SemaphoreType.DMA,
        ],
    )
    draft = sc_run(flat, cu_pad)

    zeroed = pl.pallas_call(
        _tc_zero,
        out_shape=jax.ShapeDtypeStruct((B * L, D), jnp.float32),
        in_specs=[pl.BlockSpec(memory_space=pltpu.SMEM)],
        out_specs=pl.BlockSpec(memory_space=pl.ANY),
        scratch_shapes=[
            pltpu.VMEM((G, D), jnp.float32),
            pltpu.SemaphoreType.DMA,
        ],
    )(cu)

    dense = pl.pallas_call(
        _tc_data,
        out_shape=jax.ShapeDtypeStruct((B * L, D), jnp.float32),
        in_specs=[
            pl.BlockSpec(memory_space=pltpu.SMEM),
            pl.BlockSpec(memory_space=pl.ANY),
            pl.BlockSpec(memory_space=pl.ANY),
        ],
        out_specs=pl.BlockSpec(memory_space=pl.ANY),
        scratch_shapes=[
            pltpu.VMEM((WG, D), jnp.float32),
            pltpu.VMEM((WG, D), jnp.float32),
            pltpu.VMEM((WG, D), jnp.float32),
            pltpu.VMEM((WG, D), jnp.float32),
            pltpu.SemaphoreType.DMA,
            pltpu.SemaphoreType.DMA,
            pltpu.SemaphoreType.DMA,
            pltpu.SemaphoreType.DMA,
        ],
        input_output_aliases={2: 0},
    )(cu, draft, zeroed)
    return dense.reshape(B, L, D)


# speedup vs baseline: 1.9561x; 1.1332x over previous
import jax
import jax.numpy as jnp
from jax import lax
from jax.experimental import pallas as pl
from jax.experimental.pallas import tpu as pltpu
from jax.experimental.pallas import tpu_sc as plsc

B, L, D, T = 16, 4096, 512, 32768
C = 64
LP = L + C
G = 1024
WG = G + 8
NG = L // G
NT = B * NG


def _sc_body(flat, cu_pad, draft, cu_v, buf0, buf1, rd0, rd1, wr0, wr1):
    wid = lax.axis_index("c") * 16 + lax.axis_index("s")
    b = wid // 2
    h = wid % 2
    rowbase = b * LP

    pltpu.sync_copy(cu_pad, cu_v)

    v = cu_v[pl.ds(b, 16)]
    cu_b = v[0]
    seg_len = jnp.clip(v[1] - cu_b, 0, L)
    s = lax.rem(cu_b, 8)
    a0 = cu_b - s
    nsc = (seg_len + s + C - 1) // C

    bufs = (buf0, buf1)
    rds = (rd0, rd1)
    wrs = (wr0, wr1)

    def st_of(k):
        return jnp.minimum(a0 + (2 * k + h) * C, T - C)

    def src(k):
        return flat.at[pl.ds(pl.multiple_of(st_of(k), 8), C)]

    def dst(k):
        off = rowbase + (st_of(k) - a0)
        return draft.at[pl.ds(pl.multiple_of(off, 8), C)]

    nc = jnp.clip((nsc - h + 1) // 2, 0, LP // (2 * C) + 1)

    for j in range(2):
        @pl.when(nc > j)
        def _():
            pltpu.async_copy(src(j), bufs[j], rds[j])

    def pipe_body(k2, carry):
        for j in range(2):
            k = 2 * k2 + j

            @pl.when(k < nc)
            def _():
                pltpu.make_async_copy(flat.at[pl.ds(0, C)],
                                      bufs[j], rds[j]).wait()
                pltpu.async_copy(bufs[j], dst(k), wrs[j])

                @pl.when(k + 2 < nc)
                def _():
                    pltpu.make_async_copy(bufs[j], draft.at[pl.ds(0, C)],
                                          wrs[j]).wait()
                    pltpu.async_copy(src(k + 2), bufs[j], rds[j])

        return carry

    lax.fori_loop(0, (nc + 1) // 2, pipe_body, 0)

    for j in range(2):
        @pl.when(nc > j)
        def _():
            pltpu.make_async_copy(bufs[j], draft.at[pl.ds(0, C)],
                                  wrs[j]).wait()


def _tc_zero(cu_ref, out_ref, zb, sem_z):
    zb[...] = jnp.zeros((G, D), jnp.float32)

    def row_len(bi):
        return jnp.clip(cu_ref[bi + 1] - cu_ref[bi], 0, L)

    tz = jnp.int32(0)
    for bi in range(B):
        g0 = (row_len(bi) + G - 1) // G

        def zero_body(g, carry):
            pltpu.async_copy(
                zb, out_ref.at[pl.ds(pl.multiple_of(bi * L + g * G, 8), G)],
                sem_z)
            return carry

        lax.fori_loop(g0, NG, zero_body, 0)
        tz = tz + (NG - g0)

    def drain_z(_, carry):
        pltpu.make_async_copy(zb, out_ref.at[pl.ds(0, G)], sem_z).wait()
        return carry

    lax.fori_loop(0, tz, drain_z, 0)


def _tc_data(cu_ref, draft, zeroed, out_ref, w0, w1, w2, w3,
             ob0, ob1, ob2, ob3, rd0, rd1, rd2, rd3, wr0, wr1, wr2, wr3):
    del zeroed
    ws = (w0, w1, w2, w3)
    obs = (ob0, ob1, ob2, ob3)
    rds = (rd0, rd1, rd2, rd3)
    wrs = (wr0, wr1, wr2, wr3)

    def row_len(bi):
        return jnp.clip(cu_ref[bi + 1] - cu_ref[bi], 0, L)

    def cond(t):
        return lax.rem(t, NG) < (row_len(t // NG) + G - 1) // G

    def win_ref(t):
        off = (t // NG) * LP + lax.rem(t, NG) * G
        return draft.at[pl.ds(pl.multiple_of(off, 8), WG)]

    def out_at(t):
        return out_ref.at[pl.ds(pl.multiple_of(t * G, 8), G)]

    for j in range(4):
        @pl.when(cond(j))
        def _():
            pltpu.async_copy(win_ref(j), ws[j], rds[j])

    def body(t4, carry):
        for j in range(4):
            t = 4 * t4 + j

            @pl.when(jnp.logical_and(t >= 4, cond(jnp.maximum(t - 4, 0))))
            def _():
                pltpu.make_async_copy(obs[j].at[pl.ds(0, G)],
                                      out_ref.at[pl.ds(0, G)], wrs[j]).wait()

            @pl.when(cond(t))
            def _():
                pltpu.make_async_copy(draft.at[pl.ds(0, WG)],
                                      ws[j], rds[j]).wait()
                bi = t // NG
                g = lax.rem(t, NG)
                ln = row_len(bi)
                s = lax.rem(cu_ref[bi], 8)
                for sv in range(8):
                    @pl.when(s == sv)
                    def _(sv=sv):
                        if sv == 0:
                            obs[j][...] = ws[j][...]
                        else:
                            obs[j][...] = pltpu.roll(ws[j][...], WG - sv, 0)

                partial = g * G + G > ln

                @pl.when(partial)
                def _():
                    rows = lax.broadcasted_iota(jnp.int32, (WG, D), 0) + g * G
                    obs[j][...] = jnp.where(rows < ln, obs[j][...], 0.0)

                pltpu.async_copy(obs[j].at[pl.ds(0, G)], out_at(t), wrs[j])

            tn = jnp.minimum(t + 4, NT - 1)

            @pl.when(jnp.logical_and(t + 4 < NT, cond(tn)))
            def _():
                pltpu.async_copy(win_ref(tn), ws[j], rds[j])

        return carry

    lax.fori_loop(0, NT // 4, body, 0)

    for t in (NT - 4, NT - 3, NT - 2, NT - 1):
        @pl.when(cond(jnp.int32(t)))
        def _():
            pltpu.make_async_copy(obs[t % 4].at[pl.ds(0, G)],
                                  out_ref.at[pl.ds(0, G)], wrs[t % 4]).wait()


def kernel(flat, cu_seqlens):
    cu = cu_seqlens.astype(jnp.int32)
    cu_pad = jnp.zeros((2 * B,), jnp.int32).at[:B + 1].set(cu)
    mesh = plsc.VectorSubcoreMesh(core_axis_name="c", subcore_axis_name="s")
    sc_run = pl.kernel(
        _sc_body,
        mesh=mesh,
        out_type=jax.ShapeDtypeStruct((B * LP, D), jnp.float32),
        scratch_types=[
            pltpu.VMEM((2 * B,), jnp.int32),
            pltpu.VMEM((C, D), jnp.float32),
            pltpu.VMEM((C, D), jnp.float32),
            pltpu.SemaphoreType.DMA,
            pltpu.SemaphoreType.DMA,
            pltpu.SemaphoreType.DMA,
            pltpu.SemaphoreType.DMA,
        ],
    )
    draft = sc_run(flat, cu_pad)

    zeroed = pl.pallas_call(
        _tc_zero,
        out_shape=jax.ShapeDtypeStruct((B * L, D), jnp.float32),
        in_specs=[pl.BlockSpec(memory_space=pltpu.SMEM)],
        out_specs=pl.BlockSpec(memory_space=pl.ANY),
        scratch_shapes=[
            pltpu.VMEM((G, D), jnp.float32),
            pltpu.SemaphoreType.DMA,
        ],
    )(cu)

    dense = pl.pallas_call(
        _tc_data,
        out_shape=jax.ShapeDtypeStruct((B * L, D), jnp.float32),
        in_specs=[
            pl.BlockSpec(memory_space=pltpu.SMEM),
            pl.BlockSpec(memory_space=pl.ANY),
            pl.BlockSpec(memory_space=pl.ANY),
        ],
        out_specs=pl.BlockSpec(memory_space=pl.ANY),
        scratch_shapes=(
            [pltpu.VMEM((WG, D), jnp.float32)] * 8
            + [pltpu.SemaphoreType.DMA] * 8
        ),
        input_output_aliases={2: 0},
    )(cu, draft, zeroed)
    return dense.reshape(B, L, D)
